# baseline (speedup 1.0000x reference)
import functools

import jax
import jax.numpy as jnp
from jax import lax
from jax.experimental import pallas as pl
from jax.experimental.pallas import tpu as pltpu
from jax.experimental.pallas import tpu_sc as plsc

VOCAB = 1000000
EMBED = 32
BATCH = 16384
FIELDS = 26

NW = 32
TOTAL = BATCH * FIELDS
RPW = TOTAL // NW
CHUNK = 512
NCHUNK = RPW // CHUNK
BPW = RPW // 128

_mesh = plsc.VectorSubcoreMesh(core_axis_name="c", subcore_axis_name="s")


@functools.partial(
    pl.kernel,
    mesh=_mesh,
    compiler_params=pltpu.CompilerParams(use_tc_tiling_on_sc=False,
                                         needs_layout_passes=False),
    out_type=jax.ShapeDtypeStruct((TOTAL * EMBED,), jnp.float32),
    scratch_types=[
        pltpu.VMEM((RPW,), jnp.int32),
        pltpu.VMEM((CHUNK, EMBED), jnp.float32),
        pltpu.VMEM((CHUNK, EMBED), jnp.float32),
        pltpu.VMEM((16384,), jnp.float32),
        pltpu.SemaphoreType.DMA,
        pltpu.SemaphoreType.DMA,
    ],
)
def _embed_gather(idx_hbm, table_hbm, out_hbm, idx_v, rows_a, rows_b,
                  tiles_v, sem_a, sem_b):
  wid = lax.axis_index("s") * 2 + lax.axis_index("c")
  base = wid * RPW

  pltpu.sync_copy(idx_hbm.at[pl.ds(base, RPW)], idx_v)

  lane = lax.iota(jnp.int32, 16)
  vh = [(2 * h + (lane >> 3)) * 4096 + (lane & 7) * 128 for h in range(2)]

  def fire(c, buf, sem):
    pltpu.async_copy(
        table_hbm.at[idx_v.at[pl.ds(c * CHUNK, CHUNK)]], buf, sem
    )

  def drain(buf, sem):
    pltpu.make_async_copy(table_hbm.at[pl.ds(0, CHUNK)], buf, sem).wait()

  def process(c, buf):
    def tloop(r0, carry):
      for u in range(4):
        r = r0 * 4 + u
        dst_base = (r >> 7) * 1024 + (r & 127)
        for h in range(2):
          vals = buf[r, pl.ds(16 * h, 16)]
          plsc.store_scatter(tiles_v, [vh[h] + dst_base], vals)
      return carry

    lax.fori_loop(0, CHUNK // 4, tloop, 0)

    b0 = base // 128 + c * 4
    f = b0 >> 7
    bt0 = b0 & 127
    obase = f * 4096 + bt0 * 8
    for jb in range(4):
      pltpu.sync_copy(
          tiles_v.at[pl.ds(jb * 4096, 4096)],
          out_hbm.at[pl.ds((obase + jb * 1024) * 128, 4096)],
      )

  fire(0, rows_a, sem_a)

  def body(t, carry):
    c0 = 2 * t
    fire(c0 + 1, rows_b, sem_b)
    drain(rows_a, sem_a)
    process(c0, rows_a)

    @pl.when(t < NCHUNK // 2 - 1)
    def _():
      fire(c0 + 2, rows_a, sem_a)

    drain(rows_b, sem_b)
    process(c0 + 1, rows_b)
    return carry

  lax.fori_loop(0, NCHUNK // 2, body, 0)


def kernel(inputs, embeddings):
  idx = jnp.swapaxes(inputs, 0, 1).reshape(TOTAL).astype(jnp.int32)
  out1d = _embed_gather(idx, embeddings)
  x = out1d.reshape(FIELDS, 4, 128, 8, 128)
  x = x.transpose(0, 1, 3, 2, 4)
  x = x.reshape(FIELDS, EMBED, BATCH)
  return x.transpose(2, 0, 1)


# speedup vs baseline: 1.1057x; 1.1057x over previous
import functools

import jax
import jax.numpy as jnp
from jax import lax
from jax.experimental import pallas as pl
from jax.experimental.pallas import tpu as pltpu
from jax.experimental.pallas import tpu_sc as plsc

VOCAB = 1000000
EMBED = 32
BATCH = 16384
FIELDS = 26

NW = 32
TOTAL = BATCH * FIELDS
RPW = TOTAL // NW
CHUNK = 512
NCHUNK = RPW // CHUNK
BPW = RPW // 128

_mesh = plsc.VectorSubcoreMesh(core_axis_name="c", subcore_axis_name="s")


@functools.partial(
    pl.kernel,
    mesh=_mesh,
    compiler_params=pltpu.CompilerParams(use_tc_tiling_on_sc=False,
                                         needs_layout_passes=False),
    out_type=jax.ShapeDtypeStruct((TOTAL * EMBED,), jnp.float32),
    scratch_types=[
        pltpu.VMEM((RPW,), jnp.int32),
        pltpu.VMEM((CHUNK, EMBED), jnp.float32),
        pltpu.VMEM((CHUNK, EMBED), jnp.float32),
        pltpu.VMEM((16384,), jnp.float32),
        pltpu.SemaphoreType.DMA,
        pltpu.SemaphoreType.DMA,
    ],
)
def _embed_gather(idx_hbm, table_hbm, out_hbm, idx_v, rows_a, rows_b,
                  tiles_v, sem_a, sem_b):
  wid = lax.axis_index("s") * 2 + lax.axis_index("c")
  base = wid * RPW

  pltpu.sync_copy(idx_hbm.at[pl.ds(base, RPW)], idx_v)

  lane = lax.iota(jnp.int32, 16)
  vh = [(2 * h + (lane >> 3)) * 4096 + (lane & 7) * 128 for h in range(2)]

  def fire(c, buf, sem):
    pltpu.async_copy(
        table_hbm.at[idx_v.at[pl.ds(c * CHUNK, CHUNK)]], buf, sem
    )

  def drain(buf, sem):
    pltpu.make_async_copy(table_hbm.at[pl.ds(0, CHUNK)], buf, sem).wait()

  def process(c, buf):
    @plsc.parallel_loop(0, CHUNK, 4, unroll=4)
    def tloop(r0):
      for u in range(4):
        r = r0 + u
        dst_base = (r >> 7) * 1024 + (r & 127)
        for h in range(2):
          vals = buf[r, pl.ds(16 * h, 16)]
          plsc.store_scatter(tiles_v, [vh[h] + dst_base], vals)

    b0 = base // 128 + c * 4
    f = b0 >> 7
    bt0 = b0 & 127
    obase = f * 4096 + bt0 * 8
    for jb in range(4):
      pltpu.sync_copy(
          tiles_v.at[pl.ds(jb * 4096, 4096)],
          out_hbm.at[pl.ds((obase + jb * 1024) * 128, 4096)],
      )

  fire(0, rows_a, sem_a)

  def body(t, carry):
    c0 = 2 * t
    fire(c0 + 1, rows_b, sem_b)
    drain(rows_a, sem_a)
    process(c0, rows_a)

    @pl.when(t < NCHUNK // 2 - 1)
    def _():
      fire(c0 + 2, rows_a, sem_a)

    drain(rows_b, sem_b)
    process(c0 + 1, rows_b)
    return carry

  lax.fori_loop(0, NCHUNK // 2, body, 0)


def kernel(inputs, embeddings):
  idx = jnp.swapaxes(inputs, 0, 1).reshape(TOTAL).astype(jnp.int32)
  out1d = _embed_gather(idx, embeddings)
  x = out1d.reshape(FIELDS, 4, 128, 8, 128)
  x = x.transpose(0, 1, 3, 2, 4)
  x = x.reshape(FIELDS, EMBED, BATCH)
  return x.transpose(2, 0, 1)
